# baseline (speedup 1.0000x reference)
import dataclasses
import functools

import jax
import jax.numpy as jnp
from jax import lax
from jax.experimental import pallas as pl
from jax.experimental.pallas import tpu as pltpu
from jax.experimental.pallas import tpu_sc as plsc

R = 5
D = 128
LANES = 16
NC = 1
NS = 16
NW = NC * NS
WIN = 128


def _sc_segment_sums(x, ei, typ, dst16):
    E = ei.shape[0] // 2
    chunk = E // NW
    W16 = 2 * LANES
    gld = 25
    grp = gld * W16 // LANES
    n_grp = chunk // (gld * W16)

    mesh = plsc.VectorSubcoreMesh(core_axis_name="c", subcore_axis_name="s",
                                  num_cores=NC)

    cp = pltpu.CompilerParams()
    if "needs_layout_passes" in pltpu.CompilerParams.__dataclass_fields__:
        cp = dataclasses.replace(cp, needs_layout_passes=False)

    @functools.partial(
        pl.kernel,
        compiler_params=cp,
        out_type=(
            jax.ShapeDtypeStruct((NC, R + 1, D), jnp.float32),
            jax.ShapeDtypeStruct((NW, R, LANES), jnp.float32),
        ),
        mesh=mesh,
        scratch_types=[
            pltpu.VMEM((chunk // 2,), jnp.int32),
            pltpu.VMEM((n_grp * LANES,), jnp.int32),
            pltpu.VMEM((grp * LANES,), jnp.int32),
            pltpu.VMEM((grp * LANES,), jnp.int32),
            pltpu.VMEM((grp * LANES,), jnp.int32),
            pltpu.VMEM((LANES,), jnp.int32),
            pltpu.VMEM((LANES,), jnp.int32),
            pltpu.VMEM((LANES, D), jnp.float32),
            pltpu.VMEM((R, LANES), jnp.float32),
            pltpu.VMEM((R + 1, D), jnp.float32),
            pltpu.VMEM_SHARED((R + 1, D), jnp.float32),
            pltpu.SemaphoreType.DMA,
        ],
    )
    def sc_kernel(x_hbm, ei_hbm, typ_hbm, d16_hbm, sums_hbm, cnt_hbm,
                  dstb, fb, srcb, typb, dwin, ibuf, tbuf, rowbuf, cntb, zsum,
                  acc_sum, sem):
        cid = lax.axis_index("c")
        sid = lax.axis_index("s")
        wid = sid * NC + cid
        base = wid * chunk

        cp_d = pltpu.async_copy(
            d16_hbm.at[pl.ds(wid * (chunk // 2), chunk // 2)], dstb, sem)

        @pl.when(sid == 0)
        def _():
            for r in range(R + 1):
                for j in range(D // LANES):
                    zsum[r, pl.ds(j * LANES, LANES)] = jnp.zeros(
                        (LANES,), jnp.float32)
            pltpu.sync_copy(zsum, acc_sum)

        for r in range(R):
            cntb[r, pl.ds(0, LANES)] = jnp.zeros((LANES,), jnp.float32)

        cp_d.wait()
        plsc.subcore_barrier()

        for g in range(n_grp):
            vs = [plsc.bitcast(dstb[pl.ds((g * gld + k) * LANES, LANES)],
                               jnp.int16)
                  for k in range(gld)]
            while len(vs) > 1:
                nxt = [jnp.minimum(vs[2 * i], vs[2 * i + 1])
                       for i in range(len(vs) // 2)]
                if len(vs) % 2:
                    nxt.append(vs[-1])
                vs = nxt
            fb[pl.ds(g * LANES, LANES)] = plsc.bitcast(vs[0], jnp.int32)

        def _half_zero(w):
            return ((w & jnp.int32(0xFFFF)) == 0) | (
                (w & jnp.int32(-65536)) == 0)

        vs = [plsc.bitcast(fb[pl.ds(v * LANES, LANES)], jnp.int16)
              for v in range(n_grp)]
        while len(vs) > 1:
            nxt = [jnp.minimum(vs[2 * i], vs[2 * i + 1])
                   for i in range(len(vs) // 2)]
            if len(vs) % 2:
                nxt.append(vs[-1])
            vs = nxt

        @pl.when(jnp.any(_half_zero(plsc.bitcast(vs[0], jnp.int32))))
        def _():
            @pl.loop(0, n_grp)
            def _(g):
                gbase = g * (grp * LANES)
                mv = fb[pl.ds(g * LANES, LANES)]

                @pl.when(jnp.any(_half_zero(mv)))
                def _():
                    pltpu.sync_copy(
                        ei_hbm.at[pl.ds(base + gbase, grp * LANES)], srcb)
                    pltpu.sync_copy(
                        ei_hbm.at[pl.ds(E + base + gbase, grp * LANES)],
                        dwin)
                    pltpu.sync_copy(
                        typ_hbm.at[pl.ds(base + gbase, grp * LANES)], typb)

                    @pl.loop(0, grp)
                    def _(j):
                        off = j * LANES
                        dv = dwin[pl.ds(off, LANES)]
                        m = dv == 0

                        @pl.when(jnp.any(m))
                        def _():
                            loff = j * LANES
                            tv = typb[pl.ds(loff, LANES)]
                            plsc.addupdate_scatter(
                                cntb.at[...],
                                [tv, lax.iota(jnp.int32, LANES)],
                                jnp.ones((LANES,), jnp.float32),
                                mask=m)
                            ibuf[...] = jnp.zeros((LANES,), jnp.int32)
                            tbuf[...] = jnp.full((LANES,), R, jnp.int32)
                            plsc.store_compressed(
                                ibuf.at[...], srcb[pl.ds(loff, LANES)],
                                mask=m)
                            plsc.store_compressed(
                                tbuf.at[...], typb[pl.ds(loff, LANES)],
                                mask=m)
                            pltpu.async_copy(
                                x_hbm.at[ibuf], rowbuf, sem).wait()
                            pltpu.sync_copy(
                                rowbuf, acc_sum.at[tbuf], add=True)

        plsc.subcore_barrier()

        pltpu.sync_copy(cntb, cnt_hbm.at[wid])

        @pl.when(sid == 0)
        def _():
            pltpu.sync_copy(acc_sum, sums_hbm.at[cid])

    return sc_kernel(x, ei, typ, dst16)


def _tc_head(sums_ref, cnt_ref, x0_ref, comp_ref, basis_ref, root_ref,
             bias_ref, wg_ref, bg_ref, ws_ref, bs_ref, og_ref, os_ref):
    hi = jax.lax.Precision.HIGHEST
    sums = jnp.sum(sums_ref[...], axis=0)
    cnt = jnp.sum(jnp.sum(cnt_ref[...], axis=0), axis=1, keepdims=True)
    c = jnp.maximum(cnt, 1.0)
    h = sums[:R, :] / c
    p = lax.dot_general(comp_ref[...], h, (((0,), (0,)), ((), ())),
                        precision=hi)
    conv = jnp.dot(x0_ref[...], root_ref[...], precision=hi) + bias_ref[...]
    for b in range(R):
        conv = conv + jnp.dot(p[b:b + 1, :], basis_ref[b * D:(b + 1) * D, :],
                              precision=hi)
    x1 = jnp.maximum(conv, 0.0)

    lg = lax.dot_general(x1, wg_ref[...], (((1,), (1,)), ((), ())),
                         precision=hi) + bg_ref[...]
    mg = jnp.max(lg)
    og_ref[...] = lg - mg - jnp.log(jnp.sum(jnp.exp(lg - mg)))

    ls = lax.dot_general(x1, ws_ref[...], (((1,), (1,)), ((), ())),
                         precision=hi) + bs_ref[...]
    ms = jnp.max(ls)
    os_ref[...] = ls - ms - jnp.log(jnp.sum(jnp.exp(ls - ms)))


def kernel(batch_x, batch_edge_index, batch_edge_type, comp, basis, root,
           bias, w_global, b_global, w_sense, b_sense):
    x = batch_x.astype(jnp.float32)
    ei = batch_edge_index.astype(jnp.int32).reshape(-1)
    typ = batch_edge_type.astype(jnp.int32)
    dpack = lax.bitcast_convert_type(
        batch_edge_index[1].astype(jnp.int16).reshape(-1, 2), jnp.int32)

    sums_p, cnt_p = _sc_segment_sums(x, ei, typ, dpack)

    n_global = w_global.shape[0]
    n_sense = w_sense.shape[0]
    og, os_ = pl.pallas_call(
        _tc_head,
        out_shape=(
            jax.ShapeDtypeStruct((1, n_global), jnp.float32),
            jax.ShapeDtypeStruct((1, n_sense), jnp.float32),
        ),
    )(sums_p, cnt_p, x[0:1, :], comp,
      basis.reshape(R * D, D), root,
      bias.reshape(1, D), w_global, b_global.reshape(1, n_global),
      w_sense, b_sense.reshape(1, n_sense))

    return (og.reshape(n_global), os_.reshape(n_sense))


# speedup vs baseline: 1.0039x; 1.0039x over previous
import dataclasses
import functools

import jax
import jax.numpy as jnp
from jax import lax
from jax.experimental import pallas as pl
from jax.experimental.pallas import tpu as pltpu
from jax.experimental.pallas import tpu_sc as plsc

R = 5
D = 128
LANES = 16
NC = 1
NS = 16
NW = NC * NS
WIN = 128


def _sc_segment_sums(x, ei, typ, dst16):
    E = ei.shape[0] // 2
    chunk = E // NW
    W16 = 2 * LANES
    gld = 25
    grp = gld * W16 // LANES
    n_grp = chunk // (gld * W16)

    mesh = plsc.VectorSubcoreMesh(core_axis_name="c", subcore_axis_name="s",
                                  num_cores=NC)

    cp = pltpu.CompilerParams()
    if "needs_layout_passes" in pltpu.CompilerParams.__dataclass_fields__:
        cp = dataclasses.replace(cp, needs_layout_passes=False)

    @functools.partial(
        pl.kernel,
        compiler_params=cp,
        out_type=(
            jax.ShapeDtypeStruct((NC, R + 1, D), jnp.float32),
            jax.ShapeDtypeStruct((NW, R, LANES), jnp.float32),
        ),
        mesh=mesh,
        scratch_types=[
            pltpu.VMEM((chunk // 2,), jnp.int32),
            pltpu.VMEM((n_grp * LANES,), jnp.int32),
            pltpu.VMEM((grp * LANES,), jnp.int32),
            pltpu.VMEM((grp * LANES,), jnp.int32),
            pltpu.VMEM((grp * LANES,), jnp.int32),
            pltpu.VMEM((LANES,), jnp.int32),
            pltpu.VMEM((LANES,), jnp.int32),
            pltpu.VMEM((LANES, D), jnp.float32),
            pltpu.VMEM((R, LANES), jnp.float32),
            pltpu.VMEM((R + 1, D), jnp.float32),
            pltpu.VMEM_SHARED((R + 1, D), jnp.float32),
            pltpu.SemaphoreType.DMA,
        ],
    )
    def sc_kernel(x_hbm, ei_hbm, typ_hbm, d16_hbm, sums_hbm, cnt_hbm,
                  dstb, fb, srcb, typb, dwin, ibuf, tbuf, rowbuf, cntb, zsum,
                  acc_sum, sem):
        cid = lax.axis_index("c")
        sid = lax.axis_index("s")
        wid = sid * NC + cid
        base = wid * chunk

        cp_d = pltpu.async_copy(
            d16_hbm.at[pl.ds(wid * (chunk // 2), chunk // 2)], dstb, sem)

        @pl.when(sid == 0)
        def _():
            for r in range(R + 1):
                for j in range(D // LANES):
                    zsum[r, pl.ds(j * LANES, LANES)] = jnp.zeros(
                        (LANES,), jnp.float32)
            pltpu.sync_copy(zsum, acc_sum)

        for r in range(R):
            cntb[r, pl.ds(0, LANES)] = jnp.zeros((LANES,), jnp.float32)

        cp_d.wait()
        plsc.subcore_barrier()

        @pl.loop(0, n_grp)
        def _(g):
            gb = g * gld * LANES
            vs = [plsc.bitcast(dstb[pl.ds(gb + k * LANES, LANES)],
                               jnp.int16)
                  for k in range(gld)]
            while len(vs) > 1:
                nxt = [jnp.minimum(vs[2 * i], vs[2 * i + 1])
                       for i in range(len(vs) // 2)]
                if len(vs) % 2:
                    nxt.append(vs[-1])
                vs = nxt
            fb[pl.ds(g * LANES, LANES)] = plsc.bitcast(vs[0], jnp.int32)

        def _half_zero(w):
            return ((w & jnp.int32(0xFFFF)) == 0) | (
                (w & jnp.int32(-65536)) == 0)

        vs = [plsc.bitcast(fb[pl.ds(v * LANES, LANES)], jnp.int16)
              for v in range(n_grp)]
        while len(vs) > 1:
            nxt = [jnp.minimum(vs[2 * i], vs[2 * i + 1])
                   for i in range(len(vs) // 2)]
            if len(vs) % 2:
                nxt.append(vs[-1])
            vs = nxt

        @pl.when(jnp.any(_half_zero(plsc.bitcast(vs[0], jnp.int32))))
        def _():
            @pl.loop(0, n_grp)
            def _(g):
                gbase = g * (grp * LANES)
                mv = fb[pl.ds(g * LANES, LANES)]

                @pl.when(jnp.any(_half_zero(mv)))
                def _():
                    pltpu.sync_copy(
                        ei_hbm.at[pl.ds(base + gbase, grp * LANES)], srcb)
                    pltpu.sync_copy(
                        ei_hbm.at[pl.ds(E + base + gbase, grp * LANES)],
                        dwin)
                    pltpu.sync_copy(
                        typ_hbm.at[pl.ds(base + gbase, grp * LANES)], typb)

                    @pl.loop(0, grp)
                    def _(j):
                        off = j * LANES
                        dv = dwin[pl.ds(off, LANES)]
                        m = dv == 0

                        @pl.when(jnp.any(m))
                        def _():
                            loff = j * LANES
                            tv = typb[pl.ds(loff, LANES)]
                            plsc.addupdate_scatter(
                                cntb.at[...],
                                [tv, lax.iota(jnp.int32, LANES)],
                                jnp.ones((LANES,), jnp.float32),
                                mask=m)
                            ibuf[...] = jnp.zeros((LANES,), jnp.int32)
                            tbuf[...] = jnp.full((LANES,), R, jnp.int32)
                            plsc.store_compressed(
                                ibuf.at[...], srcb[pl.ds(loff, LANES)],
                                mask=m)
                            plsc.store_compressed(
                                tbuf.at[...], typb[pl.ds(loff, LANES)],
                                mask=m)
                            pltpu.async_copy(
                                x_hbm.at[ibuf], rowbuf, sem).wait()
                            pltpu.sync_copy(
                                rowbuf, acc_sum.at[tbuf], add=True)

        plsc.subcore_barrier()

        pltpu.sync_copy(cntb, cnt_hbm.at[wid])

        @pl.when(sid == 0)
        def _():
            pltpu.sync_copy(acc_sum, sums_hbm.at[cid])

    return sc_kernel(x, ei, typ, dst16)


def _tc_head(sums_ref, cnt_ref, x0_ref, comp_ref, basis_ref, root_ref,
             bias_ref, wg_ref, bg_ref, ws_ref, bs_ref, og_ref, os_ref):
    hi = jax.lax.Precision.HIGHEST
    sums = jnp.sum(sums_ref[...], axis=0)
    cnt = jnp.sum(jnp.sum(cnt_ref[...], axis=0), axis=1, keepdims=True)
    c = jnp.maximum(cnt, 1.0)
    h = sums[:R, :] / c
    p = lax.dot_general(comp_ref[...], h, (((0,), (0,)), ((), ())),
                        precision=hi)
    conv = jnp.dot(x0_ref[...], root_ref[...], precision=hi) + bias_ref[...]
    for b in range(R):
        conv = conv + jnp.dot(p[b:b + 1, :], basis_ref[b * D:(b + 1) * D, :],
                              precision=hi)
    x1 = jnp.maximum(conv, 0.0)

    lg = lax.dot_general(x1, wg_ref[...], (((1,), (1,)), ((), ())),
                         precision=hi) + bg_ref[...]
    mg = jnp.max(lg)
    og_ref[...] = lg - mg - jnp.log(jnp.sum(jnp.exp(lg - mg)))

    ls = lax.dot_general(x1, ws_ref[...], (((1,), (1,)), ((), ())),
                         precision=hi) + bs_ref[...]
    ms = jnp.max(ls)
    os_ref[...] = ls - ms - jnp.log(jnp.sum(jnp.exp(ls - ms)))


def kernel(batch_x, batch_edge_index, batch_edge_type, comp, basis, root,
           bias, w_global, b_global, w_sense, b_sense):
    x = batch_x.astype(jnp.float32)
    ei = batch_edge_index.astype(jnp.int32).reshape(-1)
    typ = batch_edge_type.astype(jnp.int32)
    dpack = lax.bitcast_convert_type(
        batch_edge_index[1].astype(jnp.int16).reshape(-1, 2), jnp.int32)

    sums_p, cnt_p = _sc_segment_sums(x, ei, typ, dpack)

    n_global = w_global.shape[0]
    n_sense = w_sense.shape[0]
    og, os_ = pl.pallas_call(
        _tc_head,
        out_shape=(
            jax.ShapeDtypeStruct((1, n_global), jnp.float32),
            jax.ShapeDtypeStruct((1, n_sense), jnp.float32),
        ),
    )(sums_p, cnt_p, x[0:1, :], comp,
      basis.reshape(R * D, D), root,
      bias.reshape(1, D), w_global, b_global.reshape(1, n_global),
      w_sense, b_sense.reshape(1, n_sense))

    return (og.reshape(n_global), os_.reshape(n_sense))


# speedup vs baseline: 4.1160x; 4.1000x over previous
import dataclasses
import functools

import jax
import jax.numpy as jnp
from jax import lax
from jax.experimental import pallas as pl
from jax.experimental.pallas import tpu as pltpu
from jax.experimental.pallas import tpu_sc as plsc

R = 5
D = 128
LANES = 16
NC = 1
NS = 16
NW = NC * NS
WIN = 128


def _sc_segment_sums(x, ei, typ):
    E = ei.shape[0] // 2
    chunk = E // NW
    grp = 50
    n_grp = chunk // (grp * LANES)

    mesh = plsc.VectorSubcoreMesh(core_axis_name="c", subcore_axis_name="s",
                                  num_cores=NC)

    cp = pltpu.CompilerParams()
    if "needs_layout_passes" in pltpu.CompilerParams.__dataclass_fields__:
        cp = dataclasses.replace(cp, needs_layout_passes=False)

    @functools.partial(
        pl.kernel,
        compiler_params=cp,
        out_type=(
            jax.ShapeDtypeStruct((NC, R + 1, D), jnp.float32),
            jax.ShapeDtypeStruct((NW, R, LANES), jnp.float32),
        ),
        mesh=mesh,
        scratch_types=[
            pltpu.VMEM((chunk,), jnp.int32),
            pltpu.VMEM((grp * LANES,), jnp.int32),
            pltpu.VMEM((grp * LANES,), jnp.int32),
            pltpu.VMEM((LANES,), jnp.int32),
            pltpu.VMEM((LANES,), jnp.int32),
            pltpu.VMEM((LANES, D), jnp.float32),
            pltpu.VMEM((R, LANES), jnp.float32),
            pltpu.VMEM((R + 1, D), jnp.float32),
            pltpu.VMEM_SHARED((R + 1, D), jnp.float32),
            pltpu.SemaphoreType.DMA,
        ],
    )
    def sc_kernel(x_hbm, ei_hbm, typ_hbm, sums_hbm, cnt_hbm,
                  dstb, srcb, typb, ibuf, tbuf, rowbuf, cntb, zsum,
                  acc_sum, sem):
        cid = lax.axis_index("c")
        sid = lax.axis_index("s")
        wid = sid * NC + cid
        base = wid * chunk

        cp_d = pltpu.async_copy(ei_hbm.at[pl.ds(E + base, chunk)], dstb, sem)

        @pl.when(sid == 0)
        def _():
            for r in range(R + 1):
                for j in range(D // LANES):
                    zsum[r, pl.ds(j * LANES, LANES)] = jnp.zeros(
                        (LANES,), jnp.float32)
            pltpu.sync_copy(zsum, acc_sum)

        for r in range(R):
            cntb[r, pl.ds(0, LANES)] = jnp.zeros((LANES,), jnp.float32)

        cp_d.wait()
        plsc.subcore_barrier()

        @pl.loop(0, n_grp)
        def _(g):
            gbase = g * (grp * LANES)
            vs = [dstb[pl.ds(gbase + k * LANES, LANES)] for k in range(grp)]
            while len(vs) > 1:
                nxt = [jnp.minimum(vs[2 * i], vs[2 * i + 1])
                       for i in range(len(vs) // 2)]
                if len(vs) % 2:
                    nxt.append(vs[-1])
                vs = nxt

            @pl.when(jnp.any(vs[0] == 0))
            def _():
                pltpu.sync_copy(
                    ei_hbm.at[pl.ds(base + gbase, grp * LANES)], srcb)
                pltpu.sync_copy(
                    typ_hbm.at[pl.ds(base + gbase, grp * LANES)], typb)

                @pl.loop(0, grp)
                def _(j):
                    off = j * LANES
                    dv = dstb[pl.ds(gbase + off, LANES)]
                    m = dv == 0

                    @pl.when(jnp.any(m))
                    def _():
                        tv = typb[pl.ds(off, LANES)]
                        plsc.addupdate_scatter(
                            cntb.at[...],
                            [tv, lax.iota(jnp.int32, LANES)],
                            jnp.ones((LANES,), jnp.float32),
                            mask=m)
                        ibuf[...] = jnp.zeros((LANES,), jnp.int32)
                        tbuf[...] = jnp.full((LANES,), R, jnp.int32)
                        plsc.store_compressed(
                            ibuf.at[...], srcb[pl.ds(off, LANES)], mask=m)
                        plsc.store_compressed(
                            tbuf.at[...], typb[pl.ds(off, LANES)], mask=m)
                        pltpu.async_copy(
                            x_hbm.at[ibuf], rowbuf, sem).wait()
                        pltpu.sync_copy(
                            rowbuf, acc_sum.at[tbuf], add=True)

        plsc.subcore_barrier()

        pltpu.sync_copy(cntb, cnt_hbm.at[wid])

        @pl.when(sid == 0)
        def _():
            pltpu.sync_copy(acc_sum, sums_hbm.at[cid])

    return sc_kernel(x, ei, typ)


def _tc_head(sums_ref, cnt_ref, x0_ref, comp_ref, basis_ref, root_ref,
             bias_ref, wg_ref, bg_ref, ws_ref, bs_ref, og_ref, os_ref):
    hi = jax.lax.Precision.HIGHEST
    sums = jnp.sum(sums_ref[...], axis=0)
    cnt = jnp.sum(jnp.sum(cnt_ref[...], axis=0), axis=1, keepdims=True)
    c = jnp.maximum(cnt, 1.0)
    h = sums[:R, :] / c
    p = lax.dot_general(comp_ref[...], h, (((0,), (0,)), ((), ())),
                        precision=hi)
    conv = jnp.dot(x0_ref[...], root_ref[...], precision=hi) + bias_ref[...]
    for b in range(R):
        conv = conv + jnp.dot(p[b:b + 1, :], basis_ref[b * D:(b + 1) * D, :],
                              precision=hi)
    x1 = jnp.maximum(conv, 0.0)

    lg = lax.dot_general(x1, wg_ref[...], (((1,), (1,)), ((), ())),
                         precision=hi) + bg_ref[...]
    mg = jnp.max(lg)
    og_ref[...] = lg - mg - jnp.log(jnp.sum(jnp.exp(lg - mg)))

    ls = lax.dot_general(x1, ws_ref[...], (((1,), (1,)), ((), ())),
                         precision=hi) + bs_ref[...]
    ms = jnp.max(ls)
    os_ref[...] = ls - ms - jnp.log(jnp.sum(jnp.exp(ls - ms)))


def kernel(batch_x, batch_edge_index, batch_edge_type, comp, basis, root,
           bias, w_global, b_global, w_sense, b_sense):
    x = batch_x.astype(jnp.float32)
    ei = batch_edge_index.astype(jnp.int32).reshape(-1)
    typ = batch_edge_type.astype(jnp.int32)
    sums_p, cnt_p = _sc_segment_sums(x, ei, typ)

    n_global = w_global.shape[0]
    n_sense = w_sense.shape[0]
    og, os_ = pl.pallas_call(
        _tc_head,
        out_shape=(
            jax.ShapeDtypeStruct((1, n_global), jnp.float32),
            jax.ShapeDtypeStruct((1, n_sense), jnp.float32),
        ),
    )(sums_p, cnt_p, x[0:1, :], comp,
      basis.reshape(R * D, D), root,
      bias.reshape(1, D), w_global, b_global.reshape(1, n_global),
      w_sense, b_sense.reshape(1, n_sense))

    return (og.reshape(n_global), os_.reshape(n_sense))
